# baseline (speedup 1.0000x reference)
import functools

import jax
import jax.numpy as jnp
from jax import lax
from jax.experimental import pallas as pl
from jax.experimental.pallas import tpu as pltpu


def _mm_proj_body(x_ref, W_ref, rv_ref, X_ref, p_ref):
    X = jnp.dot(x_ref[...], W_ref[...], preferred_element_type=jnp.float32)
    X_ref[...] = X
    p_ref[...] = jnp.dot(X, rv_ref[...], preferred_element_type=jnp.float32)


def _matmul_proj(x, W, rv, Bn=2000):
    n, d = x.shape
    h = W.shape[1]
    grid = n // Bn
    assert grid * Bn == n
    X, p = pl.pallas_call(
        _mm_proj_body,
        grid=(grid,),
        in_specs=[
            pl.BlockSpec((Bn, d), lambda i: (i, 0)),
            pl.BlockSpec((d, h), lambda i: (0, 0)),
            pl.BlockSpec((h, 1), lambda i: (0, 0)),
        ],
        out_specs=[
            pl.BlockSpec((Bn, h), lambda i: (i, 0)),
            pl.BlockSpec((Bn, 1), lambda i: (i, 0)),
        ],
        out_shape=[
            jax.ShapeDtypeStruct((n, h), jnp.float32),
            jax.ShapeDtypeStruct((n, 1), jnp.float32),
        ],
    )(x, W, rv)
    return X, p[:, 0]


def _smoothing(Xt, hyperedges, proj):
    n = Xt.shape[0]
    pe = proj[hyperedges]
    r = jnp.arange(hyperedges.shape[0])
    u = hyperedges[r, jnp.argmax(pe, axis=1)]
    v = hyperedges[r, jnp.argmin(pe, axis=1)]
    loop = jnp.arange(n, dtype=hyperedges.dtype)
    src = jnp.concatenate([u, v, loop])
    dst = jnp.concatenate([v, u, loop])
    w = jnp.ones(src.shape[0], dtype=Xt.dtype)
    deg = jnp.zeros((n,), dtype=Xt.dtype).at[dst].add(w)
    dis = 1.0 / jnp.sqrt(jnp.clip(deg, 1e-12))
    coef = dis[dst] * w * dis[src]
    out = jnp.zeros_like(Xt).at[dst].add(coef[:, None] * Xt[src])
    return out


def kernel(x, hyperedges, W1, W2, rv1, rv2, bn_gamma, bn_beta):
    X1, proj1 = _matmul_proj(x, W1, rv1)
    X1 = _smoothing(X1, hyperedges, proj1)
    X1 = jax.nn.relu(X1)
    X1 = X1 / jnp.sqrt(1.0 + 1e-5) * bn_gamma + bn_beta
    X2 = X1 @ W2
    proj2 = (X2 @ rv2)[:, 0]
    X2 = _smoothing(X2, hyperedges, proj2)
    return jax.nn.log_softmax(X2, axis=-1)


# speedup vs baseline: 17.9441x; 17.9441x over previous
import functools

import jax
import jax.numpy as jnp
from jax import lax
from jax.experimental import pallas as pl
from jax.experimental.pallas import tpu as pltpu
from jax.experimental.pallas import tpu_sc as plsc

N = 100000
NP = 102400
EH = 100000
EHP = 100352
EPT = EHP // 32
G = 128
CAPB = 6400
_SC_MESH = dict(core_axis_name="c", subcore_axis_name="s")


def _scalar_lane(vec, i):
    return jnp.sum(jnp.where(lax.iota(jnp.int32, 16) == i, vec, jnp.int32(0)))


def _uv_body(heT, proj_hbm, u_hbm, v_hbm, proj_v, e_vs, u_v, v_v):
    wid = lax.axis_index("c") * 16 + lax.axis_index("s")
    base = wid * EPT
    pltpu.sync_copy(proj_hbm.at[pl.ds(0, N)], proj_v)
    for k in range(4):
        pltpu.sync_copy(heT.at[pl.ds(k * EHP + base, EPT)], e_vs[k])

    def body(i, carry):
        off = i * 16
        e0 = e_vs[0][pl.ds(off, 16)]
        p0 = plsc.load_gather(proj_v, [e0])
        ubest, pmax = e0, p0
        vbest, pmin = e0, p0
        for k in range(1, 4):
            ek = e_vs[k][pl.ds(off, 16)]
            pk = plsc.load_gather(proj_v, [ek])
            mx = pk > pmax
            ubest = jnp.where(mx, ek, ubest)
            pmax = jnp.where(mx, pk, pmax)
            mn = pk < pmin
            vbest = jnp.where(mn, ek, vbest)
            pmin = jnp.where(mn, pk, pmin)
        u_v[pl.ds(off, 16)] = ubest
        v_v[pl.ds(off, 16)] = vbest
        return carry

    lax.fori_loop(0, EPT // 16, body, 0)
    pltpu.sync_copy(u_v, u_hbm.at[pl.ds(base, EPT)])
    pltpu.sync_copy(v_v, v_hbm.at[pl.ds(base, EPT)])


@functools.partial(
    pl.kernel,
    out_type=[
        jax.ShapeDtypeStruct((EHP,), jnp.int32),
        jax.ShapeDtypeStruct((EHP,), jnp.int32),
    ],
    mesh=plsc.VectorSubcoreMesh(**_SC_MESH),
    scratch_types=[
        pltpu.VMEM((N,), jnp.float32),
        [pltpu.VMEM((EPT,), jnp.int32)] * 4,
        pltpu.VMEM((EPT,), jnp.int32),
        pltpu.VMEM((EPT,), jnp.int32),
    ],
    compiler_params=pltpu.CompilerParams(needs_layout_passes=False, use_tc_tiling_on_sc=False),
)
def _uv_kernel(heT, proj, u_out, v_out, proj_v, e_vs, u_v, v_v):
    _uv_body(heT, proj, u_out, v_out, proj_v, e_vs, u_v, v_v)


def _make_bin_kernel(nchunk, chrows):
    def body(u_hbm, v_hbm, bd_hbm, bs_hbm, nb_hbm, u_v, v_v, bd_vs, bs_vs, nb_v):
        wid = lax.axis_index("c") * 16 + lax.axis_index("s")
        base = wid * EPT
        pltpu.sync_copy(u_hbm.at[pl.ds(base, EPT)], u_v)
        pltpu.sync_copy(v_hbm.at[pl.ds(base, EPT)], v_v)
        iot = lax.iota(jnp.int32, 16)

        def it(i, cnts):
            off = i * 16
            uu = u_v[pl.ds(off, 16)]
            vv = v_v[pl.ds(off, 16)]
            valid = (base + off + iot) < EH
            cnts = list(cnts)
            for dd, ss in ((vv, uu), (uu, vv)):
                for c in range(nchunk):
                    lo = c * chrows
                    m = valid & (dd >= lo) & (dd < lo + chrows)
                    mi = m.astype(jnp.int32)
                    incl = plsc.cumsum(mi)
                    pos = cnts[c] + incl - mi
                    plsc.store_scatter(bd_vs[c], [pos], dd - lo, mask=m)
                    plsc.store_scatter(bs_vs[c], [pos], ss, mask=m)
                    cnts[c] = cnts[c] + jnp.max(incl)
            return tuple(cnts)

        cnts = lax.fori_loop(0, EPT // 16, it, (jnp.int32(0),) * nchunk)
        nbvec = jnp.zeros((16,), jnp.int32)
        for c in range(nchunk):
            for k in range(G // 16):
                pos = cnts[c] + k * 16 + iot
                plsc.store_scatter(bd_vs[c], [pos], chrows + iot)
                plsc.store_scatter(bs_vs[c], [pos], wid * G + k * 16 + iot)
            nb = lax.shift_right_logical(cnts[c] + (G - 1), 7)
            nbvec = jnp.where(iot == c, nb, nbvec)
        nb_v[pl.ds(0, 16)] = nbvec
        pltpu.sync_copy(nb_v, nb_hbm.at[pl.ds(wid * 16, 16)])
        for c in range(nchunk):
            pltpu.sync_copy(bd_vs[c], bd_hbm.at[pl.ds((c * 32 + wid) * CAPB, CAPB)])
            pltpu.sync_copy(bs_vs[c], bs_hbm.at[pl.ds((c * 32 + wid) * CAPB, CAPB)])

    return pl.kernel(
        body,
        out_type=[
            jax.ShapeDtypeStruct((nchunk * 32 * CAPB,), jnp.int32),
            jax.ShapeDtypeStruct((nchunk * 32 * CAPB,), jnp.int32),
            jax.ShapeDtypeStruct((512,), jnp.int32),
        ],
        mesh=plsc.VectorSubcoreMesh(**_SC_MESH),
        scratch_types=[
            pltpu.VMEM((EPT,), jnp.int32),
            pltpu.VMEM((EPT,), jnp.int32),
            [pltpu.VMEM((CAPB,), jnp.int32)] * nchunk,
            [pltpu.VMEM((CAPB,), jnp.int32)] * nchunk,
            pltpu.VMEM((16,), jnp.int32),
        ],
        compiler_params=pltpu.CompilerParams(needs_layout_passes=False, use_tc_tiling_on_sc=False),
    )


def _histlen(chrows):
    return -(-(chrows + 16) // 256) * 256


def _make_deg_kernel(nchunk, chrows, cpc):
    histlen = _histlen(chrows)
    stripe = histlen // 16

    def body(bd_hbm, nb_hbm, cnt_hbm, hist_sh, zb, dbuf, ones_v, cbuf, tmpb):
        cid = lax.axis_index("c")
        sid = lax.axis_index("s")

        def zinit(i, carry):
            zb[pl.ds(i * 16, 16)] = jnp.zeros((16,), jnp.float32)
            return carry

        lax.fori_loop(0, stripe // 16, zinit, 0)
        for k in range(G // 16):
            ones_v[pl.ds(k * 16, 16)] = jnp.full((16,), 1.0, jnp.float32)

        for kk in range(cpc):
            c = cid * cpc + kk
            pltpu.sync_copy(zb, hist_sh.at[pl.ds(sid * stripe, stripe)])
            plsc.subcore_barrier()
            for j in range(2):
                w = 2 * sid + j
                pltpu.sync_copy(nb_hbm.at[pl.ds(w * 16, 16)], cbuf)
                nb = _scalar_lane(cbuf[pl.ds(0, 16)], c)

                def blk(b, carry):
                    seg = (c * 32 + w) * CAPB + b * G
                    pltpu.sync_copy(bd_hbm.at[pl.ds(seg, G)], dbuf)
                    pltpu.sync_copy(ones_v, hist_sh.at[dbuf], add=True)
                    return carry

                lax.fori_loop(0, nb, blk, 0)
            plsc.subcore_barrier()
            pltpu.sync_copy(hist_sh.at[pl.ds(sid * stripe, stripe)], tmpb)
            pltpu.sync_copy(tmpb, cnt_hbm.at[pl.ds(c * histlen + sid * stripe, stripe)])

    return pl.kernel(
        body,
        out_type=jax.ShapeDtypeStruct((nchunk * histlen,), jnp.float32),
        mesh=plsc.VectorSubcoreMesh(**_SC_MESH),
        scratch_types=[
            pltpu.VMEM_SHARED((histlen,), jnp.float32),
            pltpu.VMEM((stripe,), jnp.float32),
            pltpu.VMEM((G,), jnp.int32),
            pltpu.VMEM((G,), jnp.float32),
            pltpu.VMEM((16,), jnp.int32),
            pltpu.VMEM((stripe,), jnp.float32),
        ],
        compiler_params=pltpu.CompilerParams(needs_layout_passes=False, use_tc_tiling_on_sc=False),
    )


def _make_row_kernel(nchunk, chrows, cpc, roww):
    accr = chrows + 16
    sr = chrows // 16

    def body(y_hbm, bd_hbm, bs_hbm, nb_hbm, zp_hbm, acc_sh, dbuf, sbuf, rows_v, cbuf):
        cid = lax.axis_index("c")
        sid = lax.axis_index("s")
        for kk in range(cpc):
            c = cid * cpc + kk
            pltpu.sync_copy(y_hbm.at[pl.ds(c * chrows + sid * sr, sr)],
                            acc_sh.at[pl.ds(sid * sr, sr)])
            plsc.subcore_barrier()
            for j in range(2):
                w = 2 * sid + j
                pltpu.sync_copy(nb_hbm.at[pl.ds(w * 16, 16)], cbuf)
                nb = _scalar_lane(cbuf[pl.ds(0, 16)], c)

                def blk(b, carry):
                    seg = (c * 32 + w) * CAPB + b * G
                    pltpu.sync_copy(bd_hbm.at[pl.ds(seg, G)], dbuf)
                    pltpu.sync_copy(bs_hbm.at[pl.ds(seg, G)], sbuf)
                    pltpu.sync_copy(y_hbm.at[sbuf], rows_v)
                    pltpu.sync_copy(rows_v, acc_sh.at[dbuf], add=True)
                    return carry

                lax.fori_loop(0, nb, blk, 0)
            plsc.subcore_barrier()
            pltpu.sync_copy(acc_sh.at[pl.ds(sid * sr, sr)],
                            zp_hbm.at[pl.ds(c * chrows + sid * sr, sr)])

    return pl.kernel(
        body,
        out_type=jax.ShapeDtypeStruct((NP, roww), jnp.float32),
        mesh=plsc.VectorSubcoreMesh(**_SC_MESH),
        scratch_types=[
            pltpu.VMEM_SHARED((accr, roww), jnp.float32),
            pltpu.VMEM((G,), jnp.int32),
            pltpu.VMEM((G,), jnp.int32),
            pltpu.VMEM((G, roww), jnp.float32),
            pltpu.VMEM((16,), jnp.int32),
        ],
        compiler_params=pltpu.CompilerParams(needs_layout_passes=False, use_tc_tiling_on_sc=False),
    )


def _mm_proj_body(x_ref, W_ref, rv_ref, X_ref, p_ref):
    X = jnp.dot(x_ref[...], W_ref[...], preferred_element_type=jnp.float32)
    X_ref[...] = X
    p_ref[...] = jnp.dot(X, rv_ref[...], preferred_element_type=jnp.float32)


def _matmul_proj(x, W, rv, Bn=800):
    n, d = x.shape
    h = W.shape[1]
    grid = n // Bn
    assert grid * Bn == n
    X, p = pl.pallas_call(
        _mm_proj_body,
        grid=(grid,),
        in_specs=[
            pl.BlockSpec((Bn, d), lambda i: (i, 0)),
            pl.BlockSpec((d, h), lambda i: (0, 0)),
            pl.BlockSpec((h, 1), lambda i: (0, 0)),
        ],
        out_specs=[
            pl.BlockSpec((Bn, h), lambda i: (i, 0)),
            pl.BlockSpec((Bn, 1), lambda i: (i, 0)),
        ],
        out_shape=[
            jax.ShapeDtypeStruct((NP, h), jnp.float32),
            jax.ShapeDtypeStruct((NP, 1), jnp.float32),
        ],
    )(x, W, rv)
    return X, p


def _scale_body(x_ref, cnt_ref, y_ref, dis_ref):
    dis = lax.rsqrt(jnp.maximum(cnt_ref[...] + 1.0, 1e-12))
    dis_ref[...] = dis
    y_ref[...] = x_ref[...] * dis


def _make_scale_kernel(roww, Bn=2048):
    return pl.pallas_call(
        _scale_body,
        grid=(NP // Bn,),
        in_specs=[
            pl.BlockSpec((Bn, roww), lambda i: (i, 0)),
            pl.BlockSpec((Bn, 1), lambda i: (i, 0)),
        ],
        out_specs=[
            pl.BlockSpec((Bn, roww), lambda i: (i, 0)),
            pl.BlockSpec((Bn, 1), lambda i: (i, 0)),
        ],
        out_shape=[
            jax.ShapeDtypeStruct((NP, roww), jnp.float32),
            jax.ShapeDtypeStruct((NP, 1), jnp.float32),
        ],
    )


def _bn_mm_body(zp_ref, dis_ref, g_ref, b_ref, W_ref, rv_ref, x2_ref, p2_ref):
    t = jnp.maximum(zp_ref[...] * dis_ref[...], 0.0)
    t = t * g_ref[...] + b_ref[...]
    X2 = jnp.dot(t, W_ref[...], preferred_element_type=jnp.float32)
    x2_ref[...] = X2
    p2_ref[...] = jnp.dot(X2, rv_ref[...], preferred_element_type=jnp.float32)


def _bn_mm(zp, dis, g2d, b2d, W2, rv2, Bn=2048):
    h, c = W2.shape
    return pl.pallas_call(
        _bn_mm_body,
        grid=(NP // Bn,),
        in_specs=[
            pl.BlockSpec((Bn, h), lambda i: (i, 0)),
            pl.BlockSpec((Bn, 1), lambda i: (i, 0)),
            pl.BlockSpec((1, h), lambda i: (0, 0)),
            pl.BlockSpec((1, h), lambda i: (0, 0)),
            pl.BlockSpec((h, c), lambda i: (0, 0)),
            pl.BlockSpec((c, 1), lambda i: (0, 0)),
        ],
        out_specs=[
            pl.BlockSpec((Bn, c), lambda i: (i, 0)),
            pl.BlockSpec((Bn, 1), lambda i: (i, 0)),
        ],
        out_shape=[
            jax.ShapeDtypeStruct((NP, c), jnp.float32),
            jax.ShapeDtypeStruct((NP, 1), jnp.float32),
        ],
    )(zp, dis, g2d, b2d, W2, rv2)


def _lsm_body(zp_ref, dis_ref, out_ref):
    L = zp_ref[...] * dis_ref[...]
    m = jnp.max(L, axis=-1, keepdims=True)
    s = jnp.log(jnp.sum(jnp.exp(L - m), axis=-1, keepdims=True))
    out_ref[...] = L - m - s


def _lsm(zp, dis, c, Bn=800):
    return pl.pallas_call(
        _lsm_body,
        grid=(N // Bn,),
        in_specs=[
            pl.BlockSpec((Bn, c), lambda i: (i, 0)),
            pl.BlockSpec((Bn, 1), lambda i: (i, 0)),
        ],
        out_specs=pl.BlockSpec((Bn, c), lambda i: (i, 0)),
        out_shape=jax.ShapeDtypeStruct((N, c), jnp.float32),
    )(zp, dis)


_L1 = dict(nchunk=8, chrows=12800, cpc=4)
_L2 = dict(nchunk=4, chrows=25600, cpc=2)
_BIN1 = _make_bin_kernel(_L1["nchunk"], _L1["chrows"])
_BIN2 = _make_bin_kernel(_L2["nchunk"], _L2["chrows"])
_DEG1 = _make_deg_kernel(**_L1)
_DEG2 = _make_deg_kernel(**_L2)
_ROW1 = _make_row_kernel(roww=128, **_L1)
_ROW2 = _make_row_kernel(roww=40, **_L2)
_SCALE128 = _make_scale_kernel(128)
_SCALE40 = _make_scale_kernel(40)


def _cnt_assemble(cnt_raw, nchunk, chrows):
    histlen = _histlen(chrows)
    return cnt_raw.reshape(nchunk, histlen)[:, :chrows].reshape(NP, 1)


def kernel(x, hyperedges, W1, W2, rv1, rv2, bn_gamma, bn_beta):
    heT = jnp.pad(hyperedges, ((0, EHP - EH), (0, 0))).T.reshape(-1)
    g2d = (bn_gamma / jnp.sqrt(1.0 + 1e-5)).reshape(1, -1)
    b2d = bn_beta.reshape(1, -1)

    X1, proj1 = _matmul_proj(x, W1, rv1)
    u1, v1 = _uv_kernel(heT, proj1.reshape(-1))
    bd1, bs1, nb1 = _BIN1(u1, v1)
    cnt1 = _cnt_assemble(_DEG1(bd1, nb1), _L1["nchunk"], _L1["chrows"])
    Y1, dis1 = _SCALE128(X1, cnt1)
    zp1 = _ROW1(Y1, bd1, bs1, nb1)

    X2, proj2 = _bn_mm(zp1, dis1, g2d, b2d, W2, rv2)
    u2, v2 = _uv_kernel(heT, proj2.reshape(-1))
    bd2, bs2, nb2 = _BIN2(u2, v2)
    cnt2 = _cnt_assemble(_DEG2(bd2, nb2), _L2["nchunk"], _L2["chrows"])
    Y2, dis2 = _SCALE40(X2, cnt2)
    zp2 = _ROW2(Y2, bd2, bs2, nb2)

    return _lsm(zp2, dis2, W2.shape[1])
